# Initial kernel scaffold; baseline (speedup 1.0000x reference)
import jax
import jax.numpy as jnp
from jax.experimental import pallas as pl


def kernel(x, edge_index, eps1, W1a, b1a, W1b, b1b, W2a, b2a, Wl, bl):
    raise NotImplementedError("write your pallas kernel here")


# speedup vs baseline: 6.6465x; 6.6465x over previous
import functools

import jax
import jax.numpy as jnp
from jax import lax
from jax.experimental import pallas as pl
from jax.experimental.pallas import tpu as pltpu
from jax.experimental.pallas import tpu_sc as plsc

N = 10000
E = 320000
D = 128

NC = 2
NS = 16
NW = NC * NS
EPW = E // NW
CHUNK = 125
NCHUNK = EPW // CHUNK
RPT = 624
REM = N - RPT * NS


def _sc_segment_sum(x, src3, dst3, zrows):
    mesh = plsc.VectorSubcoreMesh(core_axis_name="c", subcore_axis_name="s")

    @functools.partial(
        pl.kernel,
        mesh=mesh,
        out_type=jax.ShapeDtypeStruct((NC, N, D), jnp.float32),
        scratch_types=[
            pltpu.VMEM((NCHUNK, CHUNK), jnp.int32),
            pltpu.VMEM((NCHUNK, CHUNK), jnp.int32),
            pltpu.VMEM((CHUNK, D), jnp.float32),
            pltpu.VMEM_SHARED((N, D), jnp.float32),
            pltpu.SemaphoreType.DMA,
        ],
    )
    def k(x_hbm, src_hbm, dst_hbm, z_hbm, out_hbm, src_v, dst_v, rows_v, acc, sem):
        cid = lax.axis_index("c")
        sid = lax.axis_index("s")
        wid = sid * NC + cid
        pltpu.sync_copy(z_hbm, acc.at[pl.ds(sid * RPT, RPT)])

        @pl.when(sid == NS - 1)
        def _zero_rem():
            pltpu.sync_copy(z_hbm.at[pl.ds(0, REM)], acc.at[pl.ds(RPT * NS, REM)])
        pltpu.sync_copy(src_hbm.at[wid], src_v)
        pltpu.sync_copy(dst_hbm.at[wid], dst_v)
        plsc.subcore_barrier()

        def body(j, carry):
            pltpu.async_copy(x_hbm.at[src_v.at[j]], rows_v, sem).wait()
            pltpu.sync_copy(rows_v, acc.at[dst_v.at[j]], add=True)
            return carry

        lax.fori_loop(0, NCHUNK, body, 0)
        plsc.subcore_barrier()
        pltpu.sync_copy(acc.at[pl.ds(sid * RPT, RPT)],
                        out_hbm.at[cid, pl.ds(sid * RPT, RPT)])

        @pl.when(sid == NS - 1)
        def _write_rem():
            pltpu.sync_copy(acc.at[pl.ds(RPT * NS, REM)],
                            out_hbm.at[cid, pl.ds(RPT * NS, REM)])

    return k(x, src3, dst3, zrows)


_BLK = 1000


def _mlp1(x, p0, p1, scale, WaT, ba, WbT, bb):
    def body(x_r, p0_r, p1_r, s_r, wa_r, ba_r, wb_r, bb_r, o_r):
        h = x_r[...] * s_r[0, 0] + p0_r[...] + p1_r[...]
        h = jnp.maximum(
            jnp.dot(h, wa_r[...], preferred_element_type=jnp.float32,
                    precision=lax.Precision.HIGHEST) + ba_r[...], 0.0)
        h = jnp.maximum(
            jnp.dot(h, wb_r[...], preferred_element_type=jnp.float32,
                    precision=lax.Precision.HIGHEST) + bb_r[...], 0.0)
        o_r[...] = h

    row = pl.BlockSpec((_BLK, D), lambda i: (i, 0))
    full = pl.BlockSpec((D, D), lambda i: (0, 0))
    bias = pl.BlockSpec((1, D), lambda i: (0, 0))
    return pl.pallas_call(
        body,
        grid=(N // _BLK,),
        in_specs=[row, row, row, pl.BlockSpec((1, 1), lambda i: (0, 0)),
                  full, bias, full, bias],
        out_specs=row,
        out_shape=jax.ShapeDtypeStruct((N, D), jnp.float32),
    )(x, p0, p1, scale, WaT, ba, WbT, bb)


def _mlp2(h, q0, q1, WaT, ba, WlT, bl):
    def body(h_r, q0_r, q1_r, wa_r, ba_r, wl_r, bl_r, o_r):
        h2 = h_r[...] + q0_r[...] + q1_r[...]
        h2 = jnp.maximum(
            jnp.dot(h2, wa_r[...], preferred_element_type=jnp.float32,
                    precision=lax.Precision.HIGHEST) + ba_r[...], 0.0)
        o_r[...] = jnp.dot(h2, wl_r[...], preferred_element_type=jnp.float32,
                           precision=lax.Precision.HIGHEST) + bl_r[...]

    row = pl.BlockSpec((_BLK, D), lambda i: (i, 0))
    full = pl.BlockSpec((D, D), lambda i: (0, 0))
    bias = pl.BlockSpec((1, D), lambda i: (0, 0))
    return pl.pallas_call(
        body,
        grid=(N // _BLK,),
        in_specs=[row, row, row, full, bias, full, bias],
        out_specs=row,
        out_shape=jax.ShapeDtypeStruct((N, D), jnp.float32),
    )(h, q0, q1, WaT, ba, WlT, bl)


def kernel(x, edge_index, eps1, W1a, b1a, W1b, b1b, W2a, b2a, Wl, bl):
    src3 = edge_index[0].reshape(NW, NCHUNK, CHUNK)
    dst3 = edge_index[1].reshape(NW, NCHUNK, CHUNK)
    zrows = jnp.zeros((RPT, D), jnp.float32)
    scale = jnp.reshape(1.0 + eps1, (1, 1))
    p = _sc_segment_sum(x, src3, dst3, zrows)
    h = _mlp1(x, p[0], p[1], scale, W1a.T, b1a.reshape(1, D), W1b.T, b1b.reshape(1, D))
    q = _sc_segment_sum(h, src3, dst3, zrows)
    return _mlp2(h, q[0], q[1], W2a.T, b2a.reshape(1, D), Wl.T, bl.reshape(1, D))
